# baseline (speedup 1.0000x reference)
import functools

import jax
import jax.numpy as jnp
from jax import lax
from jax.experimental import pallas as pl
from jax.experimental.pallas import tpu as pltpu

N = 10000
E = 2000
H = 256
BK = 1000


def _edge0_kernel(inc_ref, x_ref, w1_ref, w2_ref, incb_ref, xw2_ref, acc_ref,
                  *, nk):
    k = pl.program_id(0)

    @pl.when(k == 0)
    def _init():
        acc_ref[...] = jnp.zeros_like(acc_ref)

    inc_blk = inc_ref[...].astype(jnp.bfloat16)
    incb_ref[...] = inc_blk
    xw1 = jnp.dot(x_ref[...], w1_ref[...], preferred_element_type=jnp.float32)
    acc_ref[...] += lax.dot_general(
        inc_blk, xw1.astype(jnp.bfloat16), (((0,), (0,)), ((), ())),
        preferred_element_type=jnp.float32)

    @pl.when(k == nk - 1)
    def _fin():
        x1 = jnp.maximum(acc_ref[...], 0.0)
        xw2_ref[...] = jnp.dot(x1, w2_ref[...], preferred_element_type=jnp.float32)


def _edge_phase_kernel(inc_ref, x_ref, w2_ref, x1_ref, xw2_ref, acc_ref, *, nk):
    k = pl.program_id(0)

    @pl.when(k == 0)
    def _init():
        acc_ref[...] = jnp.zeros_like(acc_ref)

    acc_ref[...] += lax.dot_general(
        inc_ref[...], x_ref[...].astype(jnp.bfloat16), (((0,), (0,)), ((), ())),
        preferred_element_type=jnp.float32)

    @pl.when(k == nk - 1)
    def _fin():
        x1 = jnp.maximum(acc_ref[...], 0.0)
        x1_ref[...] = x1
        xw2_ref[...] = jnp.dot(x1, w2_ref[...], preferred_element_type=jnp.float32)


def _node_phase_kernel(inc_ref, xw2_ref, w1_ref, out_ref, *, fuse_w1):
    t = jnp.maximum(
        jnp.dot(inc_ref[...], xw2_ref[...].astype(jnp.bfloat16),
                preferred_element_type=jnp.float32),
        0.0)
    if fuse_w1:
        t = jnp.dot(t, w1_ref[...], preferred_element_type=jnp.float32)
    out_ref[...] = t


def _edge0(inc, x, w1, w2):
    nk = N // BK
    return pl.pallas_call(
        functools.partial(_edge0_kernel, nk=nk),
        grid=(nk,),
        in_specs=[
            pl.BlockSpec((BK, E), lambda k: (k, 0)),
            pl.BlockSpec((BK, H), lambda k: (k, 0)),
            pl.BlockSpec((H, H), lambda k: (0, 0)),
            pl.BlockSpec((H, H), lambda k: (0, 0)),
        ],
        out_specs=[
            pl.BlockSpec((BK, E), lambda k: (k, 0)),
            pl.BlockSpec((E, H), lambda k: (0, 0)),
        ],
        out_shape=[
            jax.ShapeDtypeStruct((N, E), jnp.bfloat16),
            jax.ShapeDtypeStruct((E, H), jnp.float32),
        ],
        scratch_shapes=[pltpu.VMEM((E, H), jnp.float32)],
    )(inc, x, w1, w2)


def _edge_phase(incb, xw1, w2):
    nk = N // BK
    return pl.pallas_call(
        functools.partial(_edge_phase_kernel, nk=nk),
        grid=(nk,),
        in_specs=[
            pl.BlockSpec((BK, E), lambda k: (k, 0)),
            pl.BlockSpec((BK, H), lambda k: (k, 0)),
            pl.BlockSpec((H, H), lambda k: (0, 0)),
        ],
        out_specs=[
            pl.BlockSpec((E, H), lambda k: (0, 0)),
            pl.BlockSpec((E, H), lambda k: (0, 0)),
        ],
        out_shape=[
            jax.ShapeDtypeStruct((E, H), jnp.float32),
            jax.ShapeDtypeStruct((E, H), jnp.float32),
        ],
        scratch_shapes=[pltpu.VMEM((E, H), jnp.float32)],
    )(incb, xw1, w2)


def _node_phase(incb, xw2, w1, fuse_w1):
    nm = N // BK
    return pl.pallas_call(
        functools.partial(_node_phase_kernel, fuse_w1=fuse_w1),
        grid=(nm,),
        in_specs=[
            pl.BlockSpec((BK, E), lambda m: (m, 0)),
            pl.BlockSpec((E, H), lambda m: (0, 0)),
            pl.BlockSpec((H, H), lambda m: (0, 0)),
        ],
        out_specs=pl.BlockSpec((BK, H), lambda m: (m, 0)),
        out_shape=jax.ShapeDtypeStruct((N, H), jnp.float32),
    )(incb, xw2, w1)


def kernel(x_0, incidence_1, weight1_0, weight2_0, att_weight1_0, att_weight2_0,
           weight1_1, weight2_1, att_weight1_1, att_weight2_1):
    inc_bf, xw2_0 = _edge0(incidence_1, x_0, weight1_0, weight2_0)
    xw1_1 = _node_phase(inc_bf, xw2_0, weight1_1, fuse_w1=True)
    x1_1, xw2_1 = _edge_phase(inc_bf, xw1_1, weight2_1)
    x_out = _node_phase(inc_bf, xw2_1, weight1_1, fuse_w1=False)
    return (x_out, x1_1)


# speedup vs baseline: 1.2220x; 1.2220x over previous
import functools

import jax
import jax.numpy as jnp
from jax import lax
from jax.experimental import pallas as pl
from jax.experimental.pallas import tpu as pltpu

N = 10000
E = 2000
H = 256
BK = 1000


def _bf(x):
    return x.astype(jnp.bfloat16)


def _edge0_kernel(inc_ref, x_ref, w1_ref, w2_ref, x1_ref, xw2_ref, acc_ref,
                  *, nk):
    k = pl.program_id(0)

    @pl.when(k == 0)
    def _init():
        acc_ref[...] = jnp.zeros_like(acc_ref)

    xw1 = jnp.dot(x_ref[...], w1_ref[...], preferred_element_type=jnp.float32)
    acc_ref[...] += jnp.dot(_bf(xw1.T), inc_ref[...],
                            preferred_element_type=jnp.float32)

    @pl.when(k == nk - 1)
    def _fin():
        x1 = jnp.maximum(acc_ref[...].T, 0.0)
        x1_ref[...] = x1
        xw2_ref[...] = _bf(jnp.dot(x1, w2_ref[...],
                                   preferred_element_type=jnp.float32))


def _edge1_kernel(inc_ref, xw1_ref, w2_ref, x1_ref, xw2_ref, acc_ref, *, nk):
    k = pl.program_id(0)

    @pl.when(k == 0)
    def _init():
        acc_ref[...] = jnp.zeros_like(acc_ref)

    acc_ref[...] += jnp.dot(xw1_ref[...].T, inc_ref[...],
                            preferred_element_type=jnp.float32)

    @pl.when(k == nk - 1)
    def _fin():
        x1 = jnp.maximum(acc_ref[...].T, 0.0)
        x1_ref[...] = x1
        xw2_ref[...] = _bf(jnp.dot(x1, w2_ref[...],
                                   preferred_element_type=jnp.float32))


def _node0_kernel(inc_ref, xw2_ref, w1_ref, xw1_ref):
    t = jnp.maximum(jnp.dot(inc_ref[...], xw2_ref[...],
                            preferred_element_type=jnp.float32), 0.0)
    xw1_ref[...] = _bf(jnp.dot(t, w1_ref[...],
                               preferred_element_type=jnp.float32))


def _node1_kernel(inc_ref, xw2_ref, out_ref):
    out_ref[...] = jnp.maximum(
        jnp.dot(inc_ref[...], xw2_ref[...], preferred_element_type=jnp.float32),
        0.0)


def _edge0(inc, x, w1, w2):
    nk = N // BK
    return pl.pallas_call(
        functools.partial(_edge0_kernel, nk=nk),
        grid=(nk,),
        in_specs=[
            pl.BlockSpec((BK, E), lambda k: (k, 0)),
            pl.BlockSpec((BK, H), lambda k: (k, 0)),
            pl.BlockSpec((H, H), lambda k: (0, 0)),
            pl.BlockSpec((H, H), lambda k: (0, 0)),
        ],
        out_specs=[
            pl.BlockSpec((E, H), lambda k: (0, 0)),
            pl.BlockSpec((E, H), lambda k: (0, 0)),
        ],
        out_shape=[
            jax.ShapeDtypeStruct((E, H), jnp.float32),
            jax.ShapeDtypeStruct((E, H), jnp.bfloat16),
        ],
        scratch_shapes=[pltpu.VMEM((H, E), jnp.float32)],
    )(inc, x, w1, w2)


def _edge1(inc, xw1t, w2):
    nk = N // BK
    return pl.pallas_call(
        functools.partial(_edge1_kernel, nk=nk),
        grid=(nk,),
        in_specs=[
            pl.BlockSpec((BK, E), lambda k: (k, 0)),
            pl.BlockSpec((BK, H), lambda k: (k, 0)),
            pl.BlockSpec((H, H), lambda k: (0, 0)),
        ],
        out_specs=[
            pl.BlockSpec((E, H), lambda k: (0, 0)),
            pl.BlockSpec((E, H), lambda k: (0, 0)),
        ],
        out_shape=[
            jax.ShapeDtypeStruct((E, H), jnp.float32),
            jax.ShapeDtypeStruct((E, H), jnp.bfloat16),
        ],
        scratch_shapes=[pltpu.VMEM((H, E), jnp.float32)],
    )(inc, xw1t, w2)


def _node0(inc, xw2, w1):
    nm = N // BK
    return pl.pallas_call(
        _node0_kernel,
        grid=(nm,),
        in_specs=[
            pl.BlockSpec((BK, E), lambda m: (m, 0)),
            pl.BlockSpec((E, H), lambda m: (0, 0)),
            pl.BlockSpec((H, H), lambda m: (0, 0)),
        ],
        out_specs=pl.BlockSpec((BK, H), lambda m: (m, 0)),
        out_shape=jax.ShapeDtypeStruct((N, H), jnp.bfloat16),
    )(inc, xw2, w1)


def _node1(inc, xw2):
    nm = N // BK
    return pl.pallas_call(
        _node1_kernel,
        grid=(nm,),
        in_specs=[
            pl.BlockSpec((BK, E), lambda m: (m, 0)),
            pl.BlockSpec((E, H), lambda m: (0, 0)),
        ],
        out_specs=pl.BlockSpec((BK, H), lambda m: (m, 0)),
        out_shape=jax.ShapeDtypeStruct((N, H), jnp.float32),
    )(inc, xw2)


def kernel(x_0, incidence_1, weight1_0, weight2_0, att_weight1_0, att_weight2_0,
           weight1_1, weight2_1, att_weight1_1, att_weight2_1):
    inc_bf = incidence_1.astype(jnp.bfloat16)
    _, xw2_0 = _edge0(inc_bf, x_0, weight1_0, weight2_0)
    xw1t_1 = _node0(inc_bf, xw2_0, weight1_1)
    x1_1, xw2_1 = _edge1(inc_bf, xw1t_1, weight2_1)
    x_out = _node1(inc_bf, xw2_1)
    return (x_out, x1_1)
